# baseline (speedup 1.0000x reference)
import functools

import jax
import jax.numpy as jnp
import numpy as np
from jax import lax
from jax.experimental import pallas as pl
from jax.experimental.pallas import tpu as pltpu
from jax.experimental.pallas import tpu_sc as plsc

MAX_STEP = 3
HID_DIM = 128
HGN = 16
HGS = 10
N_NODES = 10000
N_EDGES = 320000
N_GRAPH_IDS = 128

D = HID_DIM
K = HGN * HGS
NG = N_GRAPH_IDS

NC = 2
NS = 16
NW = NC * NS
CH = 128

NP = 10240
BLK = 512
NBLK = NP // BLK
NCHUNK_G = NP // CH

EPT = N_EDGES // NW
CPT = 80
EPT_P = CPT * CH
DUMMY = NP - 8
ZPT = NP // NS
NBUF = 4

@functools.cache
def _mesh():
    return plsc.VectorSubcoreMesh(
        core_axis_name="c", subcore_axis_name="s",
        num_cores=NC, num_subcores=NS)


@functools.cache
def _sc_gather_fn():
    return pl.kernel(
        _sc_gather_body,
        out_type=jax.ShapeDtypeStruct((NP, D), jnp.float32),
        mesh=_mesh(),
        scratch_types=[
            pltpu.VMEM((CH,), jnp.int32),
            pltpu.VMEM((CH, D), jnp.float32),
            pltpu.SemaphoreType.DMA,
        ],
    )


def _sc_gather_body(idx_hbm, table_hbm, out_hbm, idx_v, rows_v, sem):
    w = lax.axis_index("c") * NS + lax.axis_index("s")
    for j in range(-(-NCHUNK_G // NW)):
        chunk = w + j * NW

        @pl.when(chunk < NCHUNK_G)
        def _():
            pltpu.sync_copy(idx_hbm.at[pl.ds(chunk * CH, CH)], idx_v)
            pltpu.async_copy(table_hbm.at[idx_v], rows_v, sem).wait()
            pltpu.sync_copy(rows_v, out_hbm.at[pl.ds(chunk * CH, CH)])


NBUF = 2
NR = CPT // NBUF


@functools.cache
def _sc_scatter_fn():
    return pl.kernel(
        _sc_scatter_body,
        out_type=jax.ShapeDtypeStruct((NC, NP, D), jnp.float32),
        mesh=_mesh(),
        scratch_types=[
            pltpu.VMEM((2 * NBUF, CH), jnp.int32),
            pltpu.VMEM((2 * NBUF, CH), jnp.int32),
            pltpu.VMEM((NBUF * CH, D), jnp.float32),
            pltpu.VMEM_SHARED((NP, D), jnp.float32),
            pltpu.SemaphoreType.DMA((2 * NBUF,)),
        ],
    )


def _sc_scatter_body(sd_hbm, xx_hbm, zeros_hbm, out_hbm,
                     srcr, dstr, rows, acc, sg):
    c = lax.axis_index("c")
    s = lax.axis_index("s")
    w = c * NS + s
    base = w * 2 * CPT
    pltpu.sync_copy(zeros_hbm, acc.at[pl.ds(s * ZPT, ZPT)])
    plsc.subcore_barrier()

    def roundfn(t, carry):
        @pl.when(t < NR)
        def _():
            pltpu.sync_copy(sd_hbm.at[pl.ds(base + t * NBUF, NBUF)],
                            srcr.at[pl.ds((t % 2) * NBUF, NBUF)])
            pltpu.sync_copy(sd_hbm.at[pl.ds(base + CPT + t * NBUF, NBUF)],
                            dstr.at[pl.ds((t % 2) * NBUF, NBUF)])

        for b in range(NBUF):
            @pl.when(t > 0)
            def _():
                pltpu.make_async_copy(xx_hbm.at[srcr.at[0, pl.ds(0, CH // 2)]],
                                      rows.at[pl.ds(0, CH // 2)],
                                      sg.at[2 * b]).wait()
                pltpu.make_async_copy(xx_hbm.at[srcr.at[0, pl.ds(0, CH // 2)]],
                                      rows.at[pl.ds(0, CH // 2)],
                                      sg.at[2 * b + 1]).wait()
                pltpu.sync_copy(rows.at[pl.ds(b * CH, CH)],
                                acc.at[dstr.at[((t + 1) % 2) * NBUF + b]],
                                add=True)

            @pl.when(t < NR)
            def _():
                pltpu.async_copy(
                    xx_hbm.at[srcr.at[(t % 2) * NBUF + b, pl.ds(0, CH // 2)]],
                    rows.at[pl.ds(b * CH, CH // 2)], sg.at[2 * b])
                pltpu.async_copy(
                    xx_hbm.at[srcr.at[(t % 2) * NBUF + b,
                                      pl.ds(CH // 2, CH // 2)]],
                    rows.at[pl.ds(b * CH + CH // 2, CH // 2)],
                    sg.at[2 * b + 1])
        return carry

    lax.fori_loop(0, NR + 1, roundfn, 0)
    plsc.subcore_barrier()
    pltpu.sync_copy(acc.at[pl.ds(s * ZPT, ZPT)],
                    out_hbm.at[c, pl.ds(s * ZPT, ZPT)])


def _dot(a, b):
    return jax.lax.dot_general(
        a, b, (((1,), (0,)), ((), ())),
        precision=jax.lax.Precision.DEFAULT,
        preferred_element_type=jnp.float32)


def _tc_a_body(pf, fcwT, fcb, z0rT, xx_out, g0_out):
    v = _dot(pf[...], fcwT[...]) + fcb[...]
    xx = jax.nn.sigmoid(v)
    xx_out[...] = xx
    g0_out[...] = _dot(xx, z0rT[...])


def _tc_a(pf, fcwT, fcb, z0rT):
    return pl.pallas_call(
        _tc_a_body,
        grid=(NBLK,),
        in_specs=[
            pl.BlockSpec((BLK, D), lambda i: (i, 0)),
            pl.BlockSpec((D, D), lambda i: (0, 0)),
            pl.BlockSpec((1, D), lambda i: (0, 0)),
            pl.BlockSpec((D, K), lambda i: (0, 0)),
        ],
        out_specs=[
            pl.BlockSpec((BLK, D), lambda i: (i, 0)),
            pl.BlockSpec((BLK, K), lambda i: (i, 0)),
        ],
        out_shape=[
            jax.ShapeDtypeStruct((NP, D), jnp.float32),
            jax.ShapeDtypeStruct((NP, K), jnp.float32),
        ],
    )(pf, fcwT, fcb, z0rT)


def _tc_b_body(p, z1rT, xx_out, g1_out):
    xx = p[0] + p[1]
    xx_out[...] = xx
    g1_out[...] = _dot(xx, z1rT[...])


def _tc_b(p, z1rT):
    return pl.pallas_call(
        _tc_b_body,
        grid=(NBLK,),
        in_specs=[
            pl.BlockSpec((NC, BLK, D), lambda i: (0, i, 0)),
            pl.BlockSpec((D, K), lambda i: (0, 0)),
        ],
        out_specs=[
            pl.BlockSpec((BLK, D), lambda i: (i, 0)),
            pl.BlockSpec((BLK, K), lambda i: (i, 0)),
        ],
        out_shape=[
            jax.ShapeDtypeStruct((NP, D), jnp.float32),
            jax.ShapeDtypeStruct((NP, K), jnp.float32),
        ],
    )(p, z1rT)


def _leaky(v):
    return jnp.where(v >= 0, v, 0.01 * v)


def _tc_final_body(p2, z2rT, g0, g1, batch3, mlpT, mlpb, out,
                   seg0, seg1, seg2):
    i = pl.program_id(0)
    g2 = _dot(p2[0] + p2[1], z2rT[...])
    g0v = g0[...]
    g1v = g1[...]
    bb = batch3[0]
    oh = (lax.broadcasted_iota(jnp.int32, (NG, BLK), 0) == bb).astype(jnp.float32)

    @pl.when(i == 0)
    def _():
        seg0[...] = jnp.zeros((NG, K), jnp.float32)
        seg1[...] = jnp.zeros((NG, K), jnp.float32)
        seg2[...] = jnp.zeros((NG, K), jnp.float32)

    seg0[...] += _dot(oh, g0v * g0v)
    seg1[...] += _dot(oh, g0v * g1v)
    seg2[...] += _dot(oh, g0v * g2)

    @pl.when(i == NBLK - 1)
    def _():
        sel = (lax.broadcasted_iota(jnp.int32, (K, HGN), 0) // HGS
               == lax.broadcasted_iota(jnp.int32, (K, HGN), 1)).astype(jnp.float32)
        u0 = _dot(seg0[...], sel)
        u1 = _dot(seg1[...], sel)
        u2 = _dot(seg2[...], sel)
        v = (_dot(u0, mlpT[0:HGN, :]) + _dot(u1, mlpT[HGN:2 * HGN, :])
             + _dot(u2, mlpT[2 * HGN:3 * HGN, :]) + mlpb[...])
        out[...] = _leaky(v)


def _tc_final(p2, z2rT, g0, g1, batch3, mlpT, mlpb):
    return pl.pallas_call(
        _tc_final_body,
        grid=(NBLK,),
        in_specs=[
            pl.BlockSpec((NC, BLK, D), lambda i: (0, i, 0)),
            pl.BlockSpec((D, K), lambda i: (0, 0)),
            pl.BlockSpec((BLK, K), lambda i: (i, 0)),
            pl.BlockSpec((BLK, K), lambda i: (i, 0)),
            pl.BlockSpec((1, 1, BLK), lambda i: (i, 0, 0)),
            pl.BlockSpec((MAX_STEP * HGN, D), lambda i: (0, 0)),
            pl.BlockSpec((1, D), lambda i: (0, 0)),
        ],
        out_specs=pl.BlockSpec((NG, D), lambda i: (0, 0)),
        out_shape=jax.ShapeDtypeStruct((NG, D), jnp.float32),
        scratch_shapes=[
            pltpu.VMEM((NG, K), jnp.float32),
            pltpu.VMEM((NG, K), jnp.float32),
            pltpu.VMEM((NG, K), jnp.float32),
        ],
    )(p2, z2rT, g0, g1, batch3, mlpT, mlpb)


def kernel(x, edge_index, batch, poi_embed_table, fc_w, fc_b,
           hidden_adj, hidden_feat, mlp_w, mlp_b):
    f32 = jnp.float32
    iu0, iu1 = np.triu_indices(HGS, 1)
    adj = jnp.zeros((HGN, HGS, HGS), f32).at[:, iu0, iu1].set(_leaky(hidden_adj))
    adj = adj + jnp.transpose(adj, (0, 2, 1))
    z0 = hidden_feat
    z1 = jnp.einsum('abc,acd->abd', adj, z0)
    z2 = jnp.einsum('abc,acd->abd', adj, z1)
    z0rT = z0.reshape(K, D).T
    z1rT = z1.reshape(K, D).T
    z2rT = z2.reshape(K, D).T
    fcwT = fc_w.T
    fcb = fc_b.reshape(1, D)
    mlpT = mlp_w.T
    mlpb = mlp_b.reshape(1, D)

    xg = jnp.pad(x.astype(jnp.int32), (0, NP - N_NODES))
    src = jnp.pad(edge_index[0].reshape(NW, EPT),
                  ((0, 0), (0, EPT_P - EPT))).reshape(NW, CPT, CH)
    dst = jnp.pad(edge_index[1].reshape(NW, EPT),
                  ((0, 0), (0, EPT_P - EPT)),
                  constant_values=DUMMY).reshape(NW, CPT, CH)
    sd = jnp.concatenate([src, dst], axis=1).reshape(NW * 2 * CPT, CH)
    batch3 = jnp.pad(batch, (0, NP - N_NODES),
                     constant_values=NG).reshape(NBLK, 1, BLK)
    zrows = jnp.zeros((ZPT, D), f32)

    pf = _sc_gather_fn()(xg, poi_embed_table)
    xx0, g0 = _tc_a(pf, fcwT, fcb, z0rT)
    p1 = _sc_scatter_fn()(sd, xx0, zrows)
    xx1, g1 = _tc_b(p1, z1rT)
    p2 = _sc_scatter_fn()(sd, xx1, zrows)
    return _tc_final(p2, z2rT, g0, g1, batch3, mlpT, mlpb)


# speedup vs baseline: 1.0093x; 1.0093x over previous
import functools

import jax
import jax.numpy as jnp
import numpy as np
from jax import lax
from jax.experimental import pallas as pl
from jax.experimental.pallas import tpu as pltpu
from jax.experimental.pallas import tpu_sc as plsc

MAX_STEP = 3
HID_DIM = 128
HGN = 16
HGS = 10
N_NODES = 10000
N_EDGES = 320000
N_GRAPH_IDS = 128

D = HID_DIM
K = HGN * HGS
NG = N_GRAPH_IDS

NC = 2
NS = 16
NW = NC * NS
CH = 128

NP = 10240
BLK = 512
NBLK = NP // BLK
NCHUNK_G = NP // CH

EPT = N_EDGES // NW
CPT = 80
EPT_P = CPT * CH
DUMMY = NP - 8
ZPT = NP // NS
NBUF = 4

@functools.cache
def _mesh():
    return plsc.VectorSubcoreMesh(
        core_axis_name="c", subcore_axis_name="s",
        num_cores=NC, num_subcores=NS)


@functools.cache
def _sc_gather_fn():
    return pl.kernel(
        _sc_gather_body,
        out_type=jax.ShapeDtypeStruct((NP, D), jnp.float32),
        mesh=_mesh(),
        scratch_types=[
            pltpu.VMEM((CH,), jnp.int32),
            pltpu.VMEM((CH, D), jnp.float32),
            pltpu.SemaphoreType.DMA,
        ],
    )


def _sc_gather_body(idx_hbm, table_hbm, out_hbm, idx_v, rows_v, sem):
    w = lax.axis_index("c") * NS + lax.axis_index("s")
    for j in range(-(-NCHUNK_G // NW)):
        chunk = w + j * NW

        @pl.when(chunk < NCHUNK_G)
        def _():
            pltpu.sync_copy(idx_hbm.at[pl.ds(chunk * CH, CH)], idx_v)
            pltpu.async_copy(table_hbm.at[idx_v], rows_v, sem).wait()
            pltpu.sync_copy(rows_v, out_hbm.at[pl.ds(chunk * CH, CH)])


NBUF = 2
NR = CPT // NBUF


@functools.cache
def _sc_scatter_fn():
    return pl.kernel(
        _sc_scatter_body,
        out_type=jax.ShapeDtypeStruct((NC, NP, D), jnp.float32),
        mesh=_mesh(),
        scratch_types=[
            pltpu.VMEM((2 * NBUF, CH), jnp.int32),
            pltpu.VMEM((2 * NBUF, CH), jnp.int32),
            pltpu.VMEM((NBUF * CH, D), jnp.float32),
            pltpu.VMEM_SHARED((NP, D), jnp.float32),
            pltpu.SemaphoreType.DMA((NBUF,)),
        ],
    )


def _sc_scatter_body(sd_hbm, xx_hbm, zeros_hbm, out_hbm,
                     srcr, dstr, rows, acc, sg):
    c = lax.axis_index("c")
    s = lax.axis_index("s")
    w = c * NS + s
    base = w * 2 * CPT
    pltpu.sync_copy(zeros_hbm, acc.at[pl.ds(s * ZPT, ZPT)])
    plsc.subcore_barrier()

    def roundfn(t, carry):
        @pl.when(t < NR)
        def _():
            pltpu.sync_copy(sd_hbm.at[pl.ds(base + t * NBUF, NBUF)],
                            srcr.at[pl.ds((t % 2) * NBUF, NBUF)])
            pltpu.sync_copy(sd_hbm.at[pl.ds(base + CPT + t * NBUF, NBUF)],
                            dstr.at[pl.ds((t % 2) * NBUF, NBUF)])

        for b in range(NBUF):
            @pl.when(t > 0)
            def _():
                pltpu.make_async_copy(xx_hbm.at[srcr.at[0]],
                                      rows.at[pl.ds(0, CH)],
                                      sg.at[b]).wait()
                pltpu.sync_copy(rows.at[pl.ds(b * CH, CH)],
                                acc.at[dstr.at[((t + 1) % 2) * NBUF + b]],
                                add=True)

            @pl.when(t < NR)
            def _():
                pltpu.async_copy(xx_hbm.at[srcr.at[(t % 2) * NBUF + b]],
                                 rows.at[pl.ds(b * CH, CH)], sg.at[b])
        return carry

    lax.fori_loop(0, NR + 1, roundfn, 0)
    plsc.subcore_barrier()
    pltpu.sync_copy(acc.at[pl.ds(s * ZPT, ZPT)],
                    out_hbm.at[c, pl.ds(s * ZPT, ZPT)])


def _dot(a, b):
    return jax.lax.dot_general(
        a, b, (((1,), (0,)), ((), ())),
        precision=jax.lax.Precision.DEFAULT,
        preferred_element_type=jnp.float32)


def _tc_a_body(pf, fcwT, fcb, z0rT, xx_out, g0_out):
    v = _dot(pf[...], fcwT[...]) + fcb[...]
    xx = jax.nn.sigmoid(v)
    xx_out[...] = xx
    g0_out[...] = _dot(xx, z0rT[...])


def _tc_a(pf, fcwT, fcb, z0rT):
    return pl.pallas_call(
        _tc_a_body,
        grid=(NBLK,),
        in_specs=[
            pl.BlockSpec((BLK, D), lambda i: (i, 0)),
            pl.BlockSpec((D, D), lambda i: (0, 0)),
            pl.BlockSpec((1, D), lambda i: (0, 0)),
            pl.BlockSpec((D, K), lambda i: (0, 0)),
        ],
        out_specs=[
            pl.BlockSpec((BLK, D), lambda i: (i, 0)),
            pl.BlockSpec((BLK, K), lambda i: (i, 0)),
        ],
        out_shape=[
            jax.ShapeDtypeStruct((NP, D), jnp.float32),
            jax.ShapeDtypeStruct((NP, K), jnp.float32),
        ],
    )(pf, fcwT, fcb, z0rT)


def _tc_add_body(p, xx_out):
    xx_out[...] = p[0] + p[1]


def _tc_add(p):
    return pl.pallas_call(
        _tc_add_body,
        grid=(NBLK,),
        in_specs=[pl.BlockSpec((NC, BLK, D), lambda i: (0, i, 0))],
        out_specs=pl.BlockSpec((BLK, D), lambda i: (i, 0)),
        out_shape=jax.ShapeDtypeStruct((NP, D), jnp.float32),
    )(p)


def _tc_mm_body(x, wt, o):
    o[...] = _dot(x[...], wt[...])


def _tc_mm(x, wt):
    return pl.pallas_call(
        _tc_mm_body,
        grid=(NBLK,),
        in_specs=[
            pl.BlockSpec((BLK, D), lambda i: (i, 0)),
            pl.BlockSpec((D, K), lambda i: (0, 0)),
        ],
        out_specs=pl.BlockSpec((BLK, K), lambda i: (i, 0)),
        out_shape=jax.ShapeDtypeStruct((NP, K), jnp.float32),
    )(x, wt)


def _leaky(v):
    return jnp.where(v >= 0, v, 0.01 * v)


def _onehot(batch3):
    bb = batch3[0]
    return (lax.broadcasted_iota(jnp.int32, (NG, BLK), 0) == bb
            ).astype(jnp.float32)


def _tc_seg_body(ga, gb, batch3, seg_out, seg):
    i = pl.program_id(0)

    @pl.when(i == 0)
    def _():
        seg[...] = jnp.zeros((NG, K), jnp.float32)

    seg[...] += _dot(_onehot(batch3), ga[...] * gb[...])

    @pl.when(i == NBLK - 1)
    def _():
        seg_out[...] = seg[...]


def _tc_seg(ga, gb, batch3):
    return pl.pallas_call(
        _tc_seg_body,
        grid=(NBLK,),
        in_specs=[
            pl.BlockSpec((BLK, K), lambda i: (i, 0)),
            pl.BlockSpec((BLK, K), lambda i: (i, 0)),
            pl.BlockSpec((1, 1, BLK), lambda i: (i, 0, 0)),
        ],
        out_specs=pl.BlockSpec((NG, K), lambda i: (0, 0)),
        out_shape=jax.ShapeDtypeStruct((NG, K), jnp.float32),
        scratch_shapes=[pltpu.VMEM((NG, K), jnp.float32)],
    )(ga, gb, batch3)


def _tc_final_body(p2, z2rT, g0, batch3, seg0, seg1, mlpT, mlpb, out, seg2):
    i = pl.program_id(0)
    g2 = _dot(p2[0] + p2[1], z2rT[...])

    @pl.when(i == 0)
    def _():
        seg2[...] = jnp.zeros((NG, K), jnp.float32)

    seg2[...] += _dot(_onehot(batch3), g0[...] * g2)

    @pl.when(i == NBLK - 1)
    def _():
        sel = (lax.broadcasted_iota(jnp.int32, (K, HGN), 0) // HGS
               == lax.broadcasted_iota(jnp.int32, (K, HGN), 1)).astype(jnp.float32)
        u0 = _dot(seg0[...], sel)
        u1 = _dot(seg1[...], sel)
        u2 = _dot(seg2[...], sel)
        v = (_dot(u0, mlpT[0:HGN, :]) + _dot(u1, mlpT[HGN:2 * HGN, :])
             + _dot(u2, mlpT[2 * HGN:3 * HGN, :]) + mlpb[...])
        out[...] = _leaky(v)


def _tc_final(p2, z2rT, g0, batch3, seg0, seg1, mlpT, mlpb):
    return pl.pallas_call(
        _tc_final_body,
        grid=(NBLK,),
        in_specs=[
            pl.BlockSpec((NC, BLK, D), lambda i: (0, i, 0)),
            pl.BlockSpec((D, K), lambda i: (0, 0)),
            pl.BlockSpec((BLK, K), lambda i: (i, 0)),
            pl.BlockSpec((1, 1, BLK), lambda i: (i, 0, 0)),
            pl.BlockSpec((NG, K), lambda i: (0, 0)),
            pl.BlockSpec((NG, K), lambda i: (0, 0)),
            pl.BlockSpec((MAX_STEP * HGN, D), lambda i: (0, 0)),
            pl.BlockSpec((1, D), lambda i: (0, 0)),
        ],
        out_specs=pl.BlockSpec((NG, D), lambda i: (0, 0)),
        out_shape=jax.ShapeDtypeStruct((NG, D), jnp.float32),
        scratch_shapes=[pltpu.VMEM((NG, K), jnp.float32)],
    )(p2, z2rT, g0, batch3, seg0, seg1, mlpT, mlpb)


def kernel(x, edge_index, batch, poi_embed_table, fc_w, fc_b,
           hidden_adj, hidden_feat, mlp_w, mlp_b):
    f32 = jnp.float32
    iu0, iu1 = np.triu_indices(HGS, 1)
    adj = jnp.zeros((HGN, HGS, HGS), f32).at[:, iu0, iu1].set(_leaky(hidden_adj))
    adj = adj + jnp.transpose(adj, (0, 2, 1))
    z0 = hidden_feat
    z1 = jnp.einsum('abc,acd->abd', adj, z0)
    z2 = jnp.einsum('abc,acd->abd', adj, z1)
    z0rT = z0.reshape(K, D).T
    z1rT = z1.reshape(K, D).T
    z2rT = z2.reshape(K, D).T
    fcwT = fc_w.T
    fcb = fc_b.reshape(1, D)
    mlpT = mlp_w.T
    mlpb = mlp_b.reshape(1, D)

    xg = jnp.pad(x.astype(jnp.int32), (0, NP - N_NODES))
    src = jnp.pad(edge_index[0].reshape(NW, EPT),
                  ((0, 0), (0, EPT_P - EPT))).reshape(NW, CPT, CH)
    dst = jnp.pad(edge_index[1].reshape(NW, EPT),
                  ((0, 0), (0, EPT_P - EPT)),
                  constant_values=DUMMY).reshape(NW, CPT, CH)
    sd = jnp.concatenate([src, dst], axis=1).reshape(NW * 2 * CPT, CH)
    batch3 = jnp.pad(batch, (0, NP - N_NODES),
                     constant_values=NG).reshape(NBLK, 1, BLK)
    zrows = jnp.zeros((ZPT, D), f32)

    pf = _sc_gather_fn()(xg, poi_embed_table)
    xx0, g0 = _tc_a(pf, fcwT, fcb, z0rT)
    p1 = _sc_scatter_fn()(sd, xx0, zrows)
    seg0 = _tc_seg(g0, g0, batch3)
    xx1 = _tc_add(p1)
    p2 = _sc_scatter_fn()(sd, xx1, zrows)
    g1 = _tc_mm(xx1, z1rT)
    seg1 = _tc_seg(g0, g1, batch3)
    return _tc_final(p2, z2rT, g0, batch3, seg0, seg1, mlpT, mlpb)


# speedup vs baseline: 1.0364x; 1.0269x over previous
import functools

import jax
import jax.numpy as jnp
import numpy as np
from jax import lax
from jax.experimental import pallas as pl
from jax.experimental.pallas import tpu as pltpu
from jax.experimental.pallas import tpu_sc as plsc

MAX_STEP = 3
HID_DIM = 128
HGN = 16
HGS = 10
N_NODES = 10000
N_EDGES = 320000
N_GRAPH_IDS = 128

D = HID_DIM
K = HGN * HGS
NG = N_GRAPH_IDS

NC = 2
NS = 16
NW = NC * NS
CH = 128

NP = 10240
BLK = 512
NBLK = NP // BLK
NCHUNK_G = NP // CH

EPT = N_EDGES // NW
CPT = 80
EPT_P = CPT * CH
DUMMY = NP - 8
ZPT = NP // NS
NBUF = 4

@functools.cache
def _mesh():
    return plsc.VectorSubcoreMesh(
        core_axis_name="c", subcore_axis_name="s",
        num_cores=NC, num_subcores=NS)


@functools.cache
def _sc_gather_fn():
    return pl.kernel(
        _sc_gather_body,
        out_type=jax.ShapeDtypeStruct((NP, D), jnp.float32),
        mesh=_mesh(),
        scratch_types=[
            pltpu.VMEM((CH,), jnp.int32),
            pltpu.VMEM((CH, D), jnp.float32),
            pltpu.SemaphoreType.DMA,
        ],
    )


def _sc_gather_body(idx_hbm, table_hbm, out_hbm, idx_v, rows_v, sem):
    w = lax.axis_index("c") * NS + lax.axis_index("s")
    for j in range(-(-NCHUNK_G // NW)):
        chunk = w + j * NW

        @pl.when(chunk < NCHUNK_G)
        def _():
            pltpu.sync_copy(idx_hbm.at[pl.ds(chunk * CH, CH)], idx_v)
            pltpu.async_copy(table_hbm.at[idx_v], rows_v, sem).wait()
            pltpu.sync_copy(rows_v, out_hbm.at[pl.ds(chunk * CH, CH)])


NBUF = 2
NR = CPT // NBUF
SB = 8


@functools.cache
def _sc_scatter_fn():
    return pl.kernel(
        _sc_scatter_body,
        out_type=jax.ShapeDtypeStruct((NC, NP, D), jnp.float32),
        mesh=_mesh(),
        scratch_types=[
            pltpu.VMEM((2 * SB * NBUF, CH), jnp.int32),
            pltpu.VMEM((2 * SB * NBUF, CH), jnp.int32),
            pltpu.VMEM((NBUF * CH, D), jnp.float32),
            pltpu.VMEM_SHARED((NP, D), jnp.float32),
            pltpu.SemaphoreType.DMA((NBUF,)),
        ],
    )


def _sc_scatter_body(sd_hbm, xx_hbm, zeros_hbm, out_hbm,
                     srcr, dstr, rows, acc, sg):
    c = lax.axis_index("c")
    s = lax.axis_index("s")
    w = c * NS + s
    base = w * 2 * CPT
    pltpu.sync_copy(zeros_hbm, acc.at[pl.ds(s * ZPT, ZPT)])
    plsc.subcore_barrier()

    def roundfn(t, carry):
        @pl.when(((t % SB) == 0) & (t < NR))
        def _():
            q = t // SB
            slot = (q % 2) * (SB * NBUF)
            pltpu.sync_copy(
                sd_hbm.at[pl.ds(base + q * (SB * NBUF), SB * NBUF)],
                srcr.at[pl.ds(slot, SB * NBUF)])
            pltpu.sync_copy(
                sd_hbm.at[pl.ds(base + CPT + q * (SB * NBUF), SB * NBUF)],
                dstr.at[pl.ds(slot, SB * NBUF)])

        tp = t - 1
        drow = (((tp // SB) % 2) * SB + tp % SB) * NBUF
        frow = (((t // SB) % 2) * SB + t % SB) * NBUF
        for b in range(NBUF):
            @pl.when(t > 0)
            def _():
                pltpu.make_async_copy(xx_hbm.at[srcr.at[0]],
                                      rows.at[pl.ds(0, CH)],
                                      sg.at[b]).wait()
                pltpu.sync_copy(rows.at[pl.ds(b * CH, CH)],
                                acc.at[dstr.at[drow + b]], add=True)

            @pl.when(t < NR)
            def _():
                pltpu.async_copy(xx_hbm.at[srcr.at[frow + b]],
                                 rows.at[pl.ds(b * CH, CH)], sg.at[b])
        return carry

    lax.fori_loop(0, NR + 1, roundfn, 0)
    plsc.subcore_barrier()
    pltpu.sync_copy(acc.at[pl.ds(s * ZPT, ZPT)],
                    out_hbm.at[c, pl.ds(s * ZPT, ZPT)])


def _dot(a, b):
    return jax.lax.dot_general(
        a, b, (((1,), (0,)), ((), ())),
        precision=jax.lax.Precision.DEFAULT,
        preferred_element_type=jnp.float32)


def _tc_a_body(pf, fcwT, fcb, z0rT, xx_out, g0_out):
    v = _dot(pf[...], fcwT[...]) + fcb[...]
    xx = jax.nn.sigmoid(v)
    xx_out[...] = xx
    g0_out[...] = _dot(xx, z0rT[...])


def _tc_a(pf, fcwT, fcb, z0rT):
    return pl.pallas_call(
        _tc_a_body,
        grid=(NBLK,),
        in_specs=[
            pl.BlockSpec((BLK, D), lambda i: (i, 0)),
            pl.BlockSpec((D, D), lambda i: (0, 0)),
            pl.BlockSpec((1, D), lambda i: (0, 0)),
            pl.BlockSpec((D, K), lambda i: (0, 0)),
        ],
        out_specs=[
            pl.BlockSpec((BLK, D), lambda i: (i, 0)),
            pl.BlockSpec((BLK, K), lambda i: (i, 0)),
        ],
        out_shape=[
            jax.ShapeDtypeStruct((NP, D), jnp.float32),
            jax.ShapeDtypeStruct((NP, K), jnp.float32),
        ],
    )(pf, fcwT, fcb, z0rT)


def _tc_add_body(p, xx_out):
    xx_out[...] = p[0] + p[1]


def _tc_add(p):
    return pl.pallas_call(
        _tc_add_body,
        grid=(NBLK,),
        in_specs=[pl.BlockSpec((NC, BLK, D), lambda i: (0, i, 0))],
        out_specs=pl.BlockSpec((BLK, D), lambda i: (i, 0)),
        out_shape=jax.ShapeDtypeStruct((NP, D), jnp.float32),
    )(p)


def _tc_mm_body(x, wt, o):
    o[...] = _dot(x[...], wt[...])


def _tc_mm(x, wt):
    return pl.pallas_call(
        _tc_mm_body,
        grid=(NBLK,),
        in_specs=[
            pl.BlockSpec((BLK, D), lambda i: (i, 0)),
            pl.BlockSpec((D, K), lambda i: (0, 0)),
        ],
        out_specs=pl.BlockSpec((BLK, K), lambda i: (i, 0)),
        out_shape=jax.ShapeDtypeStruct((NP, K), jnp.float32),
    )(x, wt)


def _leaky(v):
    return jnp.where(v >= 0, v, 0.01 * v)


def _onehot(batch3):
    bb = batch3[0]
    return (lax.broadcasted_iota(jnp.int32, (NG, BLK), 0) == bb
            ).astype(jnp.float32)


def _tc_seg_body(ga, gb, batch3, seg_out, seg):
    i = pl.program_id(0)

    @pl.when(i == 0)
    def _():
        seg[...] = jnp.zeros((NG, K), jnp.float32)

    seg[...] += _dot(_onehot(batch3), ga[...] * gb[...])

    @pl.when(i == NBLK - 1)
    def _():
        seg_out[...] = seg[...]


def _tc_seg(ga, gb, batch3):
    return pl.pallas_call(
        _tc_seg_body,
        grid=(NBLK,),
        in_specs=[
            pl.BlockSpec((BLK, K), lambda i: (i, 0)),
            pl.BlockSpec((BLK, K), lambda i: (i, 0)),
            pl.BlockSpec((1, 1, BLK), lambda i: (i, 0, 0)),
        ],
        out_specs=pl.BlockSpec((NG, K), lambda i: (0, 0)),
        out_shape=jax.ShapeDtypeStruct((NG, K), jnp.float32),
        scratch_shapes=[pltpu.VMEM((NG, K), jnp.float32)],
    )(ga, gb, batch3)


def _tc_final_body(p2, z2rT, g0, batch3, seg0, seg1, mlpT, mlpb, out, seg2):
    i = pl.program_id(0)
    g2 = _dot(p2[0] + p2[1], z2rT[...])

    @pl.when(i == 0)
    def _():
        seg2[...] = jnp.zeros((NG, K), jnp.float32)

    seg2[...] += _dot(_onehot(batch3), g0[...] * g2)

    @pl.when(i == NBLK - 1)
    def _():
        sel = (lax.broadcasted_iota(jnp.int32, (K, HGN), 0) // HGS
               == lax.broadcasted_iota(jnp.int32, (K, HGN), 1)).astype(jnp.float32)
        u0 = _dot(seg0[...], sel)
        u1 = _dot(seg1[...], sel)
        u2 = _dot(seg2[...], sel)
        v = (_dot(u0, mlpT[0:HGN, :]) + _dot(u1, mlpT[HGN:2 * HGN, :])
             + _dot(u2, mlpT[2 * HGN:3 * HGN, :]) + mlpb[...])
        out[...] = _leaky(v)


def _tc_final(p2, z2rT, g0, batch3, seg0, seg1, mlpT, mlpb):
    return pl.pallas_call(
        _tc_final_body,
        grid=(NBLK,),
        in_specs=[
            pl.BlockSpec((NC, BLK, D), lambda i: (0, i, 0)),
            pl.BlockSpec((D, K), lambda i: (0, 0)),
            pl.BlockSpec((BLK, K), lambda i: (i, 0)),
            pl.BlockSpec((1, 1, BLK), lambda i: (i, 0, 0)),
            pl.BlockSpec((NG, K), lambda i: (0, 0)),
            pl.BlockSpec((NG, K), lambda i: (0, 0)),
            pl.BlockSpec((MAX_STEP * HGN, D), lambda i: (0, 0)),
            pl.BlockSpec((1, D), lambda i: (0, 0)),
        ],
        out_specs=pl.BlockSpec((NG, D), lambda i: (0, 0)),
        out_shape=jax.ShapeDtypeStruct((NG, D), jnp.float32),
        scratch_shapes=[pltpu.VMEM((NG, K), jnp.float32)],
    )(p2, z2rT, g0, batch3, seg0, seg1, mlpT, mlpb)


def kernel(x, edge_index, batch, poi_embed_table, fc_w, fc_b,
           hidden_adj, hidden_feat, mlp_w, mlp_b):
    f32 = jnp.float32
    iu0, iu1 = np.triu_indices(HGS, 1)
    adj = jnp.zeros((HGN, HGS, HGS), f32).at[:, iu0, iu1].set(_leaky(hidden_adj))
    adj = adj + jnp.transpose(adj, (0, 2, 1))
    z0 = hidden_feat
    z1 = jnp.einsum('abc,acd->abd', adj, z0)
    z2 = jnp.einsum('abc,acd->abd', adj, z1)
    z0rT = z0.reshape(K, D).T
    z1rT = z1.reshape(K, D).T
    z2rT = z2.reshape(K, D).T
    fcwT = fc_w.T
    fcb = fc_b.reshape(1, D)
    mlpT = mlp_w.T
    mlpb = mlp_b.reshape(1, D)

    xg = jnp.pad(x.astype(jnp.int32), (0, NP - N_NODES))
    src = jnp.pad(edge_index[0].reshape(NW, EPT),
                  ((0, 0), (0, EPT_P - EPT))).reshape(NW, CPT, CH)
    dst = jnp.pad(edge_index[1].reshape(NW, EPT),
                  ((0, 0), (0, EPT_P - EPT)),
                  constant_values=DUMMY).reshape(NW, CPT, CH)
    sd = jnp.concatenate([src, dst], axis=1).reshape(NW * 2 * CPT, CH)
    batch3 = jnp.pad(batch, (0, NP - N_NODES),
                     constant_values=NG).reshape(NBLK, 1, BLK)
    zrows = jnp.zeros((ZPT, D), f32)

    pf = _sc_gather_fn()(xg, poi_embed_table)
    xx0, g0 = _tc_a(pf, fcwT, fcb, z0rT)
    p1 = _sc_scatter_fn()(sd, xx0, zrows)
    seg0 = _tc_seg(g0, g0, batch3)
    xx1 = _tc_add(p1)
    p2 = _sc_scatter_fn()(sd, xx1, zrows)
    g1 = _tc_mm(xx1, z1rT)
    seg1 = _tc_seg(g0, g1, batch3)
    return _tc_final(p2, z2rT, g0, batch3, seg0, seg1, mlpT, mlpb)


# speedup vs baseline: 1.0400x; 1.0035x over previous
import functools

import jax
import jax.numpy as jnp
import numpy as np
from jax import lax
from jax.experimental import pallas as pl
from jax.experimental.pallas import tpu as pltpu
from jax.experimental.pallas import tpu_sc as plsc

MAX_STEP = 3
HID_DIM = 128
HGN = 16
HGS = 10
N_NODES = 10000
N_EDGES = 320000
N_GRAPH_IDS = 128

D = HID_DIM
K = HGN * HGS
NG = N_GRAPH_IDS

NC = 2
NS = 16
NW = NC * NS
CH = 128

NP = 10240
BLK = 512
NBLK = NP // BLK
NCHUNK_G = NP // CH

EPT = N_EDGES // NW
CPT = 80
EPT_P = CPT * CH
DUMMY = NP - 8
ZPT = NP // NS
NBUF = 4

@functools.cache
def _mesh():
    return plsc.VectorSubcoreMesh(
        core_axis_name="c", subcore_axis_name="s",
        num_cores=NC, num_subcores=NS)


@functools.cache
def _sc_gather_fn():
    return pl.kernel(
        _sc_gather_body,
        out_type=jax.ShapeDtypeStruct((NP, D), jnp.float32),
        mesh=_mesh(),
        scratch_types=[
            pltpu.VMEM((3 * CH,), jnp.int32),
            pltpu.VMEM((CH, D), jnp.float32),
            pltpu.VMEM((CH, D), jnp.float32),
            pltpu.SemaphoreType.DMA((2,)),
        ],
    )


def _sc_gather_body(idx_hbm, table_hbm, out_hbm, idx_v, r0, r1, sem):
    w = lax.axis_index("c") * NS + lax.axis_index("s")
    nj = -(-NCHUNK_G // NW)
    rows = (r0, r1)
    for j in range(nj):
        @pl.when(w + j * NW < NCHUNK_G)
        def _():
            pltpu.sync_copy(idx_hbm.at[pl.ds((w + j * NW) * CH, CH)],
                            idx_v.at[pl.ds(j * CH, CH)])

    @pl.when(w < NCHUNK_G)
    def _():
        pltpu.async_copy(table_hbm.at[idx_v.at[pl.ds(0, CH)]], r0, sem.at[0])

    for j in range(nj):
        chunk = w + j * NW

        @pl.when(chunk < NCHUNK_G)
        def _():
            pltpu.make_async_copy(table_hbm.at[idx_v.at[pl.ds(0, CH)]],
                                  rows[j % 2], sem.at[j % 2]).wait()
            if j + 1 < nj:
                @pl.when(chunk + NW < NCHUNK_G)
                def _():
                    pltpu.async_copy(
                        table_hbm.at[idx_v.at[pl.ds((j + 1) * CH, CH)]],
                        rows[(j + 1) % 2], sem.at[(j + 1) % 2])

            pltpu.sync_copy(rows[j % 2], out_hbm.at[pl.ds(chunk * CH, CH)])


NBUF = 2
NR = CPT // NBUF
SB = 8


@functools.cache
def _sc_scatter_fn():
    return pl.kernel(
        _sc_scatter_body,
        out_type=jax.ShapeDtypeStruct((NC, NP, D), jnp.float32),
        mesh=_mesh(),
        scratch_types=[
            pltpu.VMEM((2 * SB * NBUF, CH), jnp.int32),
            pltpu.VMEM((2 * SB * NBUF, CH), jnp.int32),
            pltpu.VMEM((NBUF * CH, D), jnp.float32),
            pltpu.VMEM_SHARED((NP, D), jnp.float32),
            pltpu.SemaphoreType.DMA((NBUF,)),
        ],
    )


def _sc_scatter_body(sd_hbm, xx_hbm, zeros_hbm, out_hbm,
                     srcr, dstr, rows, acc, sg):
    c = lax.axis_index("c")
    s = lax.axis_index("s")
    w = c * NS + s
    base = w * 2 * CPT

    def roundfn(t, carry):
        @pl.when(((t % SB) == 0) & (t < NR))
        def _():
            q = t // SB
            slot = (q % 2) * (SB * NBUF)
            pltpu.sync_copy(
                sd_hbm.at[pl.ds(base + q * (SB * NBUF), SB * NBUF)],
                srcr.at[pl.ds(slot, SB * NBUF)])
            pltpu.sync_copy(
                sd_hbm.at[pl.ds(base + CPT + q * (SB * NBUF), SB * NBUF)],
                dstr.at[pl.ds(slot, SB * NBUF)])

        @pl.when(t == 1)
        def _():
            pltpu.sync_copy(zeros_hbm, acc.at[pl.ds(s * ZPT, ZPT)])
            plsc.subcore_barrier()

        tp = t - 1
        drow = (((tp // SB) % 2) * SB + tp % SB) * NBUF
        frow = (((t // SB) % 2) * SB + t % SB) * NBUF
        for b in range(NBUF):
            @pl.when(t > 0)
            def _():
                pltpu.make_async_copy(xx_hbm.at[srcr.at[0]],
                                      rows.at[pl.ds(0, CH)],
                                      sg.at[b]).wait()
                pltpu.sync_copy(rows.at[pl.ds(b * CH, CH)],
                                acc.at[dstr.at[drow + b]], add=True)

            @pl.when(t < NR)
            def _():
                pltpu.async_copy(xx_hbm.at[srcr.at[frow + b]],
                                 rows.at[pl.ds(b * CH, CH)], sg.at[b])
        return carry

    lax.fori_loop(0, NR + 1, roundfn, 0)
    plsc.subcore_barrier()
    pltpu.sync_copy(acc.at[pl.ds(s * ZPT, ZPT)],
                    out_hbm.at[c, pl.ds(s * ZPT, ZPT)])


def _dot(a, b):
    return jax.lax.dot_general(
        a, b, (((1,), (0,)), ((), ())),
        precision=jax.lax.Precision.DEFAULT,
        preferred_element_type=jnp.float32)


def _tc_a_body(pf, fcwT, fcb, z0rT, xx_out, g0_out):
    v = _dot(pf[...], fcwT[...]) + fcb[...]
    xx = jax.nn.sigmoid(v)
    xx_out[...] = xx
    g0_out[...] = _dot(xx, z0rT[...])


def _tc_a(pf, fcwT, fcb, z0rT):
    return pl.pallas_call(
        _tc_a_body,
        grid=(NBLK,),
        in_specs=[
            pl.BlockSpec((BLK, D), lambda i: (i, 0)),
            pl.BlockSpec((D, D), lambda i: (0, 0)),
            pl.BlockSpec((1, D), lambda i: (0, 0)),
            pl.BlockSpec((D, K), lambda i: (0, 0)),
        ],
        out_specs=[
            pl.BlockSpec((BLK, D), lambda i: (i, 0)),
            pl.BlockSpec((BLK, K), lambda i: (i, 0)),
        ],
        out_shape=[
            jax.ShapeDtypeStruct((NP, D), jnp.float32),
            jax.ShapeDtypeStruct((NP, K), jnp.float32),
        ],
    )(pf, fcwT, fcb, z0rT)


def _tc_add_body(p, xx_out):
    xx_out[...] = p[0] + p[1]


def _tc_add(p):
    return pl.pallas_call(
        _tc_add_body,
        grid=(NBLK,),
        in_specs=[pl.BlockSpec((NC, BLK, D), lambda i: (0, i, 0))],
        out_specs=pl.BlockSpec((BLK, D), lambda i: (i, 0)),
        out_shape=jax.ShapeDtypeStruct((NP, D), jnp.float32),
    )(p)


def _tc_mm_body(x, wt, o):
    o[...] = _dot(x[...], wt[...])


def _tc_mm(x, wt):
    return pl.pallas_call(
        _tc_mm_body,
        grid=(NBLK,),
        in_specs=[
            pl.BlockSpec((BLK, D), lambda i: (i, 0)),
            pl.BlockSpec((D, K), lambda i: (0, 0)),
        ],
        out_specs=pl.BlockSpec((BLK, K), lambda i: (i, 0)),
        out_shape=jax.ShapeDtypeStruct((NP, K), jnp.float32),
    )(x, wt)


def _leaky(v):
    return jnp.where(v >= 0, v, 0.01 * v)


def _onehot(batch3):
    bb = batch3[0]
    return (lax.broadcasted_iota(jnp.int32, (NG, BLK), 0) == bb
            ).astype(jnp.float32)


def _tc_seg_body(ga, gb, batch3, seg_out, seg):
    i = pl.program_id(0)

    @pl.when(i == 0)
    def _():
        seg[...] = jnp.zeros((NG, K), jnp.float32)

    seg[...] += _dot(_onehot(batch3), ga[...] * gb[...])

    @pl.when(i == NBLK - 1)
    def _():
        seg_out[...] = seg[...]


def _tc_seg(ga, gb, batch3):
    return pl.pallas_call(
        _tc_seg_body,
        grid=(NBLK,),
        in_specs=[
            pl.BlockSpec((BLK, K), lambda i: (i, 0)),
            pl.BlockSpec((BLK, K), lambda i: (i, 0)),
            pl.BlockSpec((1, 1, BLK), lambda i: (i, 0, 0)),
        ],
        out_specs=pl.BlockSpec((NG, K), lambda i: (0, 0)),
        out_shape=jax.ShapeDtypeStruct((NG, K), jnp.float32),
        scratch_shapes=[pltpu.VMEM((NG, K), jnp.float32)],
    )(ga, gb, batch3)


def _tc_final_body(p2, z2rT, g0, batch3, seg0, seg1, mlpT, mlpb, out, seg2):
    i = pl.program_id(0)
    g2 = _dot(p2[0] + p2[1], z2rT[...])

    @pl.when(i == 0)
    def _():
        seg2[...] = jnp.zeros((NG, K), jnp.float32)

    seg2[...] += _dot(_onehot(batch3), g0[...] * g2)

    @pl.when(i == NBLK - 1)
    def _():
        sel = (lax.broadcasted_iota(jnp.int32, (K, HGN), 0) // HGS
               == lax.broadcasted_iota(jnp.int32, (K, HGN), 1)).astype(jnp.float32)
        u0 = _dot(seg0[...], sel)
        u1 = _dot(seg1[...], sel)
        u2 = _dot(seg2[...], sel)
        v = (_dot(u0, mlpT[0:HGN, :]) + _dot(u1, mlpT[HGN:2 * HGN, :])
             + _dot(u2, mlpT[2 * HGN:3 * HGN, :]) + mlpb[...])
        out[...] = _leaky(v)


def _tc_final(p2, z2rT, g0, batch3, seg0, seg1, mlpT, mlpb):
    return pl.pallas_call(
        _tc_final_body,
        grid=(NBLK,),
        in_specs=[
            pl.BlockSpec((NC, BLK, D), lambda i: (0, i, 0)),
            pl.BlockSpec((D, K), lambda i: (0, 0)),
            pl.BlockSpec((BLK, K), lambda i: (i, 0)),
            pl.BlockSpec((1, 1, BLK), lambda i: (i, 0, 0)),
            pl.BlockSpec((NG, K), lambda i: (0, 0)),
            pl.BlockSpec((NG, K), lambda i: (0, 0)),
            pl.BlockSpec((MAX_STEP * HGN, D), lambda i: (0, 0)),
            pl.BlockSpec((1, D), lambda i: (0, 0)),
        ],
        out_specs=pl.BlockSpec((NG, D), lambda i: (0, 0)),
        out_shape=jax.ShapeDtypeStruct((NG, D), jnp.float32),
        scratch_shapes=[pltpu.VMEM((NG, K), jnp.float32)],
    )(p2, z2rT, g0, batch3, seg0, seg1, mlpT, mlpb)


def kernel(x, edge_index, batch, poi_embed_table, fc_w, fc_b,
           hidden_adj, hidden_feat, mlp_w, mlp_b):
    f32 = jnp.float32
    iu0, iu1 = np.triu_indices(HGS, 1)
    adj = jnp.zeros((HGN, HGS, HGS), f32).at[:, iu0, iu1].set(_leaky(hidden_adj))
    adj = adj + jnp.transpose(adj, (0, 2, 1))
    z0 = hidden_feat
    z1 = jnp.einsum('abc,acd->abd', adj, z0)
    z2 = jnp.einsum('abc,acd->abd', adj, z1)
    z0rT = z0.reshape(K, D).T
    z1rT = z1.reshape(K, D).T
    z2rT = z2.reshape(K, D).T
    fcwT = fc_w.T
    fcb = fc_b.reshape(1, D)
    mlpT = mlp_w.T
    mlpb = mlp_b.reshape(1, D)

    xg = jnp.pad(x.astype(jnp.int32), (0, NP - N_NODES))
    src = jnp.pad(edge_index[0].reshape(NW, EPT),
                  ((0, 0), (0, EPT_P - EPT))).reshape(NW, CPT, CH)
    dst = jnp.pad(edge_index[1].reshape(NW, EPT),
                  ((0, 0), (0, EPT_P - EPT)),
                  constant_values=DUMMY).reshape(NW, CPT, CH)
    sd = jnp.concatenate([src, dst], axis=1).reshape(NW * 2 * CPT, CH)
    batch3 = jnp.pad(batch, (0, NP - N_NODES),
                     constant_values=NG).reshape(NBLK, 1, BLK)
    zrows = jnp.zeros((ZPT, D), f32)

    pf = _sc_gather_fn()(xg, poi_embed_table)
    xx0, g0 = _tc_a(pf, fcwT, fcb, z0rT)
    p1 = _sc_scatter_fn()(sd, xx0, zrows)
    seg0 = _tc_seg(g0, g0, batch3)
    xx1 = _tc_add(p1)
    p2 = _sc_scatter_fn()(sd, xx1, zrows)
    g1 = _tc_mm(xx1, z1rT)
    seg1 = _tc_seg(g0, g1, batch3)
    return _tc_final(p2, z2rT, g0, batch3, seg0, seg1, mlpT, mlpb)


# speedup vs baseline: 1.0482x; 1.0079x over previous
import functools

import jax
import jax.numpy as jnp
import numpy as np
from jax import lax
from jax.experimental import pallas as pl
from jax.experimental.pallas import tpu as pltpu
from jax.experimental.pallas import tpu_sc as plsc

MAX_STEP = 3
HID_DIM = 128
HGN = 16
HGS = 10
N_NODES = 10000
N_EDGES = 320000
N_GRAPH_IDS = 128

D = HID_DIM
K = HGN * HGS
NG = N_GRAPH_IDS

NC = 2
NS = 16
NW = NC * NS
CH = 128

NP = 10240
BLK = 512
NBLK = NP // BLK
NCHUNK_G = NP // CH

EPT = N_EDGES // NW
CPT = 80
EPT_P = CPT * CH
DUMMY = NP - 8
ZPT = NP // NS
NBUF = 4

@functools.cache
def _mesh():
    return plsc.VectorSubcoreMesh(
        core_axis_name="c", subcore_axis_name="s",
        num_cores=NC, num_subcores=NS)


@functools.cache
def _sc_gather_fn():
    return pl.kernel(
        _sc_gather_body,
        out_type=jax.ShapeDtypeStruct((NP, D), jnp.float32),
        mesh=_mesh(),
        scratch_types=[
            pltpu.VMEM((3 * CH,), jnp.int32),
            pltpu.VMEM((CH, D), jnp.float32),
            pltpu.VMEM((CH, D), jnp.float32),
            pltpu.SemaphoreType.DMA((2,)),
        ],
    )


def _sc_gather_body(idx_hbm, table_hbm, out_hbm, idx_v, r0, r1, sem):
    w = lax.axis_index("c") * NS + lax.axis_index("s")
    nj = -(-NCHUNK_G // NW)
    rows = (r0, r1)
    for j in range(nj):
        @pl.when(w + j * NW < NCHUNK_G)
        def _():
            pltpu.sync_copy(idx_hbm.at[pl.ds((w + j * NW) * CH, CH)],
                            idx_v.at[pl.ds(j * CH, CH)])

    @pl.when(w < NCHUNK_G)
    def _():
        pltpu.async_copy(table_hbm.at[idx_v.at[pl.ds(0, CH)]], r0, sem.at[0])

    for j in range(nj):
        chunk = w + j * NW

        @pl.when(chunk < NCHUNK_G)
        def _():
            pltpu.make_async_copy(table_hbm.at[idx_v.at[pl.ds(0, CH)]],
                                  rows[j % 2], sem.at[j % 2]).wait()
            if j + 1 < nj:
                @pl.when(chunk + NW < NCHUNK_G)
                def _():
                    pltpu.async_copy(
                        table_hbm.at[idx_v.at[pl.ds((j + 1) * CH, CH)]],
                        rows[(j + 1) % 2], sem.at[(j + 1) % 2])

            pltpu.sync_copy(rows[j % 2], out_hbm.at[pl.ds(chunk * CH, CH)])


NBUF = 2
NR = CPT // NBUF
SB = 8


@functools.cache
def _sc_scatter_fn():
    return pl.kernel(
        _sc_scatter_body,
        out_type=jax.ShapeDtypeStruct((NC, NP, D), jnp.float32),
        mesh=_mesh(),
        scratch_types=[
            pltpu.VMEM((2 * SB * NBUF, CH), jnp.int32),
            pltpu.VMEM((2 * SB * NBUF, CH), jnp.int32),
            pltpu.VMEM((NBUF * CH, D), jnp.float32),
            pltpu.VMEM_SHARED((NP, D), jnp.float32),
            pltpu.SemaphoreType.DMA((NBUF,)),
            pltpu.SemaphoreType.DMA((2,)),
        ],
    )


def _sc_scatter_body(sd_hbm, xx_hbm, zeros_hbm, out_hbm,
                     srcr, dstr, rows, acc, sg, stg):
    c = lax.axis_index("c")
    s = lax.axis_index("s")
    w = c * NS + s
    base = w * 2 * CPT

    def roundfn(t, carry):
        @pl.when(t == 0)
        def _():
            pltpu.sync_copy(sd_hbm.at[pl.ds(base, SB * NBUF)],
                            srcr.at[pl.ds(0, SB * NBUF)])
            pltpu.sync_copy(sd_hbm.at[pl.ds(base + CPT, SB * NBUF)],
                            dstr.at[pl.ds(0, SB * NBUF)])

        @pl.when(((t % SB) == 1) & (t + SB < NR + 1))
        def _():
            qn = t // SB + 1
            slot = (qn % 2) * (SB * NBUF)
            pltpu.async_copy(
                sd_hbm.at[pl.ds(base + qn * (SB * NBUF), SB * NBUF)],
                srcr.at[pl.ds(slot, SB * NBUF)], stg.at[0])
            pltpu.async_copy(
                sd_hbm.at[pl.ds(base + CPT + qn * (SB * NBUF), SB * NBUF)],
                dstr.at[pl.ds(slot, SB * NBUF)], stg.at[1])

        @pl.when(((t % SB) == 0) & (t > 0) & (t < NR))
        def _():
            pltpu.make_async_copy(sd_hbm.at[pl.ds(base, SB * NBUF)],
                                  srcr.at[pl.ds(0, SB * NBUF)],
                                  stg.at[0]).wait()
            pltpu.make_async_copy(sd_hbm.at[pl.ds(base, SB * NBUF)],
                                  dstr.at[pl.ds(0, SB * NBUF)],
                                  stg.at[1]).wait()

        @pl.when(t == 1)
        def _():
            pltpu.sync_copy(zeros_hbm, acc.at[pl.ds(s * ZPT, ZPT)])
            plsc.subcore_barrier()

        tp = t - 1
        drow = (((tp // SB) % 2) * SB + tp % SB) * NBUF
        frow = (((t // SB) % 2) * SB + t % SB) * NBUF
        for b in range(NBUF):
            @pl.when(t > 0)
            def _():
                pltpu.make_async_copy(xx_hbm.at[srcr.at[0]],
                                      rows.at[pl.ds(0, CH)],
                                      sg.at[b]).wait()
                pltpu.sync_copy(rows.at[pl.ds(b * CH, CH)],
                                acc.at[dstr.at[drow + b]], add=True)

            @pl.when(t < NR)
            def _():
                pltpu.async_copy(xx_hbm.at[srcr.at[frow + b]],
                                 rows.at[pl.ds(b * CH, CH)], sg.at[b])
        return carry

    lax.fori_loop(0, NR + 1, roundfn, 0)
    plsc.subcore_barrier()
    pltpu.sync_copy(acc.at[pl.ds(s * ZPT, ZPT)],
                    out_hbm.at[c, pl.ds(s * ZPT, ZPT)])


def _dot(a, b):
    return jax.lax.dot_general(
        a, b, (((1,), (0,)), ((), ())),
        precision=jax.lax.Precision.DEFAULT,
        preferred_element_type=jnp.float32)


def _tc_a_body(pf, fcwT, fcb, z0rT, xx_out, g0_out):
    v = _dot(pf[...], fcwT[...]) + fcb[...]
    xx = jax.nn.sigmoid(v)
    xx_out[...] = xx
    g0_out[...] = _dot(xx, z0rT[...])


def _tc_a(pf, fcwT, fcb, z0rT):
    return pl.pallas_call(
        _tc_a_body,
        grid=(NBLK,),
        in_specs=[
            pl.BlockSpec((BLK, D), lambda i: (i, 0)),
            pl.BlockSpec((D, D), lambda i: (0, 0)),
            pl.BlockSpec((1, D), lambda i: (0, 0)),
            pl.BlockSpec((D, K), lambda i: (0, 0)),
        ],
        out_specs=[
            pl.BlockSpec((BLK, D), lambda i: (i, 0)),
            pl.BlockSpec((BLK, K), lambda i: (i, 0)),
        ],
        out_shape=[
            jax.ShapeDtypeStruct((NP, D), jnp.float32),
            jax.ShapeDtypeStruct((NP, K), jnp.float32),
        ],
    )(pf, fcwT, fcb, z0rT)


def _tc_add_body(p, xx_out):
    xx_out[...] = p[0] + p[1]


def _tc_add(p):
    return pl.pallas_call(
        _tc_add_body,
        grid=(NBLK,),
        in_specs=[pl.BlockSpec((NC, BLK, D), lambda i: (0, i, 0))],
        out_specs=pl.BlockSpec((BLK, D), lambda i: (i, 0)),
        out_shape=jax.ShapeDtypeStruct((NP, D), jnp.float32),
    )(p)


def _tc_mm_body(x, wt, o):
    o[...] = _dot(x[...], wt[...])


def _tc_mm(x, wt):
    return pl.pallas_call(
        _tc_mm_body,
        grid=(NBLK,),
        in_specs=[
            pl.BlockSpec((BLK, D), lambda i: (i, 0)),
            pl.BlockSpec((D, K), lambda i: (0, 0)),
        ],
        out_specs=pl.BlockSpec((BLK, K), lambda i: (i, 0)),
        out_shape=jax.ShapeDtypeStruct((NP, K), jnp.float32),
    )(x, wt)


def _leaky(v):
    return jnp.where(v >= 0, v, 0.01 * v)


def _onehot(batch3):
    bb = batch3[0]
    return (lax.broadcasted_iota(jnp.int32, (NG, BLK), 0) == bb
            ).astype(jnp.float32)


def _tc_seg_body(ga, gb, batch3, seg_out, seg):
    i = pl.program_id(0)

    @pl.when(i == 0)
    def _():
        seg[...] = jnp.zeros((NG, K), jnp.float32)

    seg[...] += _dot(_onehot(batch3), ga[...] * gb[...])

    @pl.when(i == NBLK - 1)
    def _():
        seg_out[...] = seg[...]


def _tc_seg(ga, gb, batch3):
    return pl.pallas_call(
        _tc_seg_body,
        grid=(NBLK,),
        in_specs=[
            pl.BlockSpec((BLK, K), lambda i: (i, 0)),
            pl.BlockSpec((BLK, K), lambda i: (i, 0)),
            pl.BlockSpec((1, 1, BLK), lambda i: (i, 0, 0)),
        ],
        out_specs=pl.BlockSpec((NG, K), lambda i: (0, 0)),
        out_shape=jax.ShapeDtypeStruct((NG, K), jnp.float32),
        scratch_shapes=[pltpu.VMEM((NG, K), jnp.float32)],
    )(ga, gb, batch3)


def _tc_final_body(p2, z2rT, g0, batch3, seg0, seg1, mlpT, mlpb, out, seg2):
    i = pl.program_id(0)
    g2 = _dot(p2[0] + p2[1], z2rT[...])

    @pl.when(i == 0)
    def _():
        seg2[...] = jnp.zeros((NG, K), jnp.float32)

    seg2[...] += _dot(_onehot(batch3), g0[...] * g2)

    @pl.when(i == NBLK - 1)
    def _():
        sel = (lax.broadcasted_iota(jnp.int32, (K, HGN), 0) // HGS
               == lax.broadcasted_iota(jnp.int32, (K, HGN), 1)).astype(jnp.float32)
        u0 = _dot(seg0[...], sel)
        u1 = _dot(seg1[...], sel)
        u2 = _dot(seg2[...], sel)
        v = (_dot(u0, mlpT[0:HGN, :]) + _dot(u1, mlpT[HGN:2 * HGN, :])
             + _dot(u2, mlpT[2 * HGN:3 * HGN, :]) + mlpb[...])
        out[...] = _leaky(v)


def _tc_final(p2, z2rT, g0, batch3, seg0, seg1, mlpT, mlpb):
    return pl.pallas_call(
        _tc_final_body,
        grid=(NBLK,),
        in_specs=[
            pl.BlockSpec((NC, BLK, D), lambda i: (0, i, 0)),
            pl.BlockSpec((D, K), lambda i: (0, 0)),
            pl.BlockSpec((BLK, K), lambda i: (i, 0)),
            pl.BlockSpec((1, 1, BLK), lambda i: (i, 0, 0)),
            pl.BlockSpec((NG, K), lambda i: (0, 0)),
            pl.BlockSpec((NG, K), lambda i: (0, 0)),
            pl.BlockSpec((MAX_STEP * HGN, D), lambda i: (0, 0)),
            pl.BlockSpec((1, D), lambda i: (0, 0)),
        ],
        out_specs=pl.BlockSpec((NG, D), lambda i: (0, 0)),
        out_shape=jax.ShapeDtypeStruct((NG, D), jnp.float32),
        scratch_shapes=[pltpu.VMEM((NG, K), jnp.float32)],
    )(p2, z2rT, g0, batch3, seg0, seg1, mlpT, mlpb)


def kernel(x, edge_index, batch, poi_embed_table, fc_w, fc_b,
           hidden_adj, hidden_feat, mlp_w, mlp_b):
    f32 = jnp.float32
    iu0, iu1 = np.triu_indices(HGS, 1)
    adj = jnp.zeros((HGN, HGS, HGS), f32).at[:, iu0, iu1].set(_leaky(hidden_adj))
    adj = adj + jnp.transpose(adj, (0, 2, 1))
    z0 = hidden_feat
    z1 = jnp.einsum('abc,acd->abd', adj, z0)
    z2 = jnp.einsum('abc,acd->abd', adj, z1)
    z0rT = z0.reshape(K, D).T
    z1rT = z1.reshape(K, D).T
    z2rT = z2.reshape(K, D).T
    fcwT = fc_w.T
    fcb = fc_b.reshape(1, D)
    mlpT = mlp_w.T
    mlpb = mlp_b.reshape(1, D)

    xg = jnp.pad(x.astype(jnp.int32), (0, NP - N_NODES))
    src = jnp.pad(edge_index[0].reshape(NW, EPT),
                  ((0, 0), (0, EPT_P - EPT))).reshape(NW, CPT, CH)
    dst = jnp.pad(edge_index[1].reshape(NW, EPT),
                  ((0, 0), (0, EPT_P - EPT)),
                  constant_values=DUMMY).reshape(NW, CPT, CH)
    sd = jnp.concatenate([src, dst], axis=1).reshape(NW * 2 * CPT, CH)
    batch3 = jnp.pad(batch, (0, NP - N_NODES),
                     constant_values=NG).reshape(NBLK, 1, BLK)
    zrows = jnp.zeros((ZPT, D), f32)

    pf = _sc_gather_fn()(xg, poi_embed_table)
    xx0, g0 = _tc_a(pf, fcwT, fcb, z0rT)
    p1 = _sc_scatter_fn()(sd, xx0, zrows)
    seg0 = _tc_seg(g0, g0, batch3)
    xx1 = _tc_add(p1)
    p2 = _sc_scatter_fn()(sd, xx1, zrows)
    g1 = _tc_mm(xx1, z1rT)
    seg1 = _tc_seg(g0, g1, batch3)
    return _tc_final(p2, z2rT, g0, batch3, seg0, seg1, mlpT, mlpb)
